# Initial kernel scaffold; baseline (speedup 1.0000x reference)
import jax
import jax.numpy as jnp
from jax.experimental import pallas as pl


def kernel(user_indices, item_indices, user_table, item_table, W1, b1):
    raise NotImplementedError("write your pallas kernel here")


# speedup vs baseline: 5.5444x; 5.5444x over previous
import functools

import jax
import jax.numpy as jnp
from jax import lax
from jax.experimental import pallas as pl
from jax.experimental.pallas import tpu as pltpu
from jax.experimental.pallas import tpu_sc as plsc

BATCH = 16384
D = 128
NC = 2
NS = 16
NW = NC * NS
CHUNK = 128
ROWS_PER_W = BATCH // NW
CHUNKS_PER_W = ROWS_PER_W // CHUNK


def _sc_gather(user_idx2, item_idx2, user_table, item_table):
    mesh = plsc.VectorSubcoreMesh(
        core_axis_name="c", subcore_axis_name="s", num_cores=NC, num_subcores=NS
    )

    @functools.partial(
        pl.kernel,
        mesh=mesh,
        out_type=(
            jax.ShapeDtypeStruct((BATCH, D), jnp.float32),
            jax.ShapeDtypeStruct((BATCH, D), jnp.float32),
        ),
        scratch_types=[
            pltpu.VMEM((CHUNK,), jnp.int32),
            pltpu.VMEM((CHUNK, D), jnp.float32),
            pltpu.SemaphoreType.DMA,
        ],
    )
    def gather_kernel(ui_hbm, ii_hbm, ut_hbm, it_hbm, u_out, i_out, idx_v, rows_v, sem):
        wid = lax.axis_index("s") * NC + lax.axis_index("c")
        for j in range(CHUNKS_PER_W):
            r = wid * CHUNKS_PER_W + j
            pltpu.sync_copy(ui_hbm.at[r], idx_v)
            pltpu.async_copy(ut_hbm.at[idx_v], rows_v, sem).wait()
            pltpu.sync_copy(rows_v, u_out.at[pl.ds(r * CHUNK, CHUNK)])
        for j in range(CHUNKS_PER_W):
            r = wid * CHUNKS_PER_W + j
            pltpu.sync_copy(ii_hbm.at[r], idx_v)
            pltpu.async_copy(it_hbm.at[idx_v], rows_v, sem).wait()
            pltpu.sync_copy(rows_v, i_out.at[pl.ds(r * CHUNK, CHUNK)])

    return gather_kernel(user_idx2, item_idx2, user_table, item_table)


def _tc_mlp(u_rows, i_rows, Wt, b2):
    BM = 2048

    def body(u_ref, i_ref, wt_ref, b_ref, o_ref):
        acc = jnp.dot(u_ref[...], wt_ref[0:D, :], preferred_element_type=jnp.float32)
        acc += jnp.dot(i_ref[...], wt_ref[D : 2 * D, :], preferred_element_type=jnp.float32)
        acc += b_ref[...]
        o_ref[...] = jnp.maximum(acc, 0.0)

    return pl.pallas_call(
        body,
        grid=(BATCH // BM,),
        in_specs=[
            pl.BlockSpec((BM, D), lambda i: (i, 0)),
            pl.BlockSpec((BM, D), lambda i: (i, 0)),
            pl.BlockSpec((2 * D, D), lambda i: (0, 0)),
            pl.BlockSpec((1, D), lambda i: (0, 0)),
        ],
        out_specs=pl.BlockSpec((BM, D), lambda i: (i, 0)),
        out_shape=jax.ShapeDtypeStruct((BATCH, D), jnp.float32),
    )(u_rows, i_rows, Wt, b2)


def kernel(user_indices, item_indices, user_table, item_table, W1, b1):
    ui2 = user_indices.reshape(BATCH // CHUNK, CHUNK)
    ii2 = item_indices.reshape(BATCH // CHUNK, CHUNK)
    u_rows, i_rows = _sc_gather(ui2, ii2, user_table, item_table)
    Wt = W1.T
    b2 = b1.reshape(1, D)
    return _tc_mlp(u_rows, i_rows, Wt, b2)


# speedup vs baseline: 6.6278x; 1.1954x over previous
import functools

import jax
import jax.numpy as jnp
from jax import lax
from jax.experimental import pallas as pl
from jax.experimental.pallas import tpu as pltpu
from jax.experimental.pallas import tpu_sc as plsc

BATCH = 16384
D = 128
NC = 2
NS = 16
NW = NC * NS
CHUNK = 128
ROWS_PER_W = BATCH // NW
CHUNKS_PER_W = ROWS_PER_W // CHUNK


def _sc_gather(user_idx2, item_idx2, user_table, item_table):
    mesh = plsc.VectorSubcoreMesh(
        core_axis_name="c", subcore_axis_name="s", num_cores=NC, num_subcores=NS
    )

    @functools.partial(
        pl.kernel,
        mesh=mesh,
        out_type=(
            jax.ShapeDtypeStruct((BATCH, D), jnp.float32),
            jax.ShapeDtypeStruct((BATCH, D), jnp.float32),
        ),
        scratch_types=[
            pltpu.VMEM((CHUNKS_PER_W, CHUNK), jnp.int32),
            pltpu.VMEM((CHUNKS_PER_W, CHUNK), jnp.int32),
            pltpu.VMEM((4, CHUNK, D), jnp.float32),
            pltpu.SemaphoreType.DMA((4,)),
            pltpu.SemaphoreType.DMA((4,)),
        ],
    )
    def gather_kernel(ui_hbm, ii_hbm, ut_hbm, it_hbm, u_out, i_out,
                      idx_u, idx_i, bufs, gsem, wsem):
        wid = lax.axis_index("s") * NC + lax.axis_index("c")
        base = wid * CHUNKS_PER_W
        pltpu.sync_copy(ui_hbm.at[pl.ds(base, CHUNKS_PER_W)], idx_u)
        pltpu.sync_copy(ii_hbm.at[pl.ds(base, CHUNKS_PER_W)], idx_i)

        NT = 2 * CHUNKS_PER_W

        def chunk(t):
            j = t % CHUNKS_PER_W
            if t < CHUNKS_PER_W:
                return ut_hbm, idx_u.at[j], u_out, j
            return it_hbm, idx_i.at[j], i_out, j

        ghandles = [None] * NT
        whandles = [None] * NT
        for t in range(NT):
            m = t % 4
            if t >= 4:
                whandles[t - 4].wait()
            table, idxref, out, j = chunk(t)
            ghandles[t] = pltpu.async_copy(table.at[idxref], bufs.at[m], gsem.at[m])
            if t >= 3:
                tt = t - 3
                mm = tt % 4
                tbl2, _, out2, j2 = chunk(tt)
                ghandles[tt].wait()
                whandles[tt] = pltpu.async_copy(
                    bufs.at[mm], out2.at[pl.ds((base + j2) * CHUNK, CHUNK)], wsem.at[mm]
                )
        for tt in range(NT - 3, NT):
            mm = tt % 4
            _, _, out2, j2 = chunk(tt)
            ghandles[tt].wait()
            whandles[tt] = pltpu.async_copy(
                bufs.at[mm], out2.at[pl.ds((base + j2) * CHUNK, CHUNK)], wsem.at[mm]
            )
        for tt in range(NT - 4, NT):
            whandles[tt].wait()

    return gather_kernel(user_idx2, item_idx2, user_table, item_table)


def _tc_mlp(u_rows, i_rows, Wt, b2):
    BM = 2048

    def body(u_ref, i_ref, wt_ref, b_ref, o_ref):
        acc = jnp.dot(u_ref[...], wt_ref[0:D, :], preferred_element_type=jnp.float32)
        acc += jnp.dot(i_ref[...], wt_ref[D : 2 * D, :], preferred_element_type=jnp.float32)
        acc += b_ref[...]
        o_ref[...] = jnp.maximum(acc, 0.0)

    return pl.pallas_call(
        body,
        grid=(BATCH // BM,),
        in_specs=[
            pl.BlockSpec((BM, D), lambda i: (i, 0)),
            pl.BlockSpec((BM, D), lambda i: (i, 0)),
            pl.BlockSpec((2 * D, D), lambda i: (0, 0)),
            pl.BlockSpec((1, D), lambda i: (0, 0)),
        ],
        out_specs=pl.BlockSpec((BM, D), lambda i: (i, 0)),
        out_shape=jax.ShapeDtypeStruct((BATCH, D), jnp.float32),
    )(u_rows, i_rows, Wt, b2)


def kernel(user_indices, item_indices, user_table, item_table, W1, b1):
    ui2 = user_indices.reshape(BATCH // CHUNK, CHUNK)
    ii2 = item_indices.reshape(BATCH // CHUNK, CHUNK)
    u_rows, i_rows = _sc_gather(ui2, ii2, user_table, item_table)
    Wt = W1.T
    b2 = b1.reshape(1, D)
    return _tc_mlp(u_rows, i_rows, Wt, b2)
